# baseline (speedup 1.0000x reference)
import jax
import jax.numpy as jnp
from jax import lax
from jax.experimental import pallas as pl
from jax.experimental.pallas import tpu as pltpu
from jax.experimental.pallas import tpu_sc as plsc

_SIZE = 1000000
_DIM = 64
_MOM = 0.9
_B = 16384

_NC = 2
_NS = 16
_L = 16
_NW = _NC * _NS
_BPW = _B // _NW
_CH = 128
_NCH = _BPW // _CH

_W = 2 * _DIM
_N = _SIZE + 1
_TBS = 40960
_TBS2 = _TBS // 2
_NB = -(-_N // _TBS)
_NPAD = _NB * _TBS

_mesh = plsc.VectorSubcoreMesh(core_axis_name="c", subcore_axis_name="s")


def _t_fwd_body(src_ref, dst_ref):
  v = src_ref[...]
  dst_ref[:, 0:_DIM] = v[:, 0:_TBS2].T
  dst_ref[:, _DIM:_W] = v[:, _TBS2:_TBS].T


def _t_bwd_body(src_ref, dst_ref):
  v = src_ref[...]
  dst_ref[:, 0:_TBS2] = v[:, 0:_DIM].T
  dst_ref[:, _TBS2:_TBS] = v[:, _DIM:_W].T


_t_fwd = pl.pallas_call(
    _t_fwd_body,
    grid=(_NB,),
    in_specs=[pl.BlockSpec((_DIM, _TBS), lambda i: (0, i))],
    out_specs=pl.BlockSpec((_TBS2, _W), lambda i: (i, 0)),
    out_shape=jax.ShapeDtypeStruct((_NPAD // 2, _W), jnp.float32),
)

_t_bwd = pl.pallas_call(
    _t_bwd_body,
    grid=(_NB,),
    in_specs=[pl.BlockSpec((_TBS2, _W), lambda i: (i, 0))],
    out_specs=pl.BlockSpec((_DIM, _TBS), lambda i: (0, i)),
    out_shape=jax.ShapeDtypeStruct((_DIM, _N), jnp.float32),
)


def _gather_body(idx_hbm, x_hbm, mem_ref, da_out, ne_out, idx_v, rows_a,
                 rows_b, x_v, sem_a, sem_b):
  wid = lax.axis_index("s") * _NC + lax.axis_index("c")
  base = wid * _BPW
  for j in range(_NCH):
    pltpu.sync_copy(idx_hbm.at[pl.ds(base + j * _CH, _CH)], idx_v.at[j])
  rows = (rows_a, rows_b)
  sems = (sem_a, sem_b)
  copies = [pltpu.async_copy(mem_ref.at[idx_v.at[0]], rows[0], sems[0])]
  for j in range(_NCH):
    if j + 1 < _NCH:
      copies.append(pltpu.async_copy(
          mem_ref.at[idx_v.at[j + 1]], rows[(j + 1) % 2], sems[(j + 1) % 2]))
    row0 = base + j * _CH
    pltpu.sync_copy(x_hbm.at[pl.ds(row0, _CH)], x_v)
    copies[j].wait()
    rows_v = rows[j % 2]
    pltpu.sync_copy(rows_v, da_out.at[pl.ds(row0, _CH)])

    @pl.loop(0, _CH)
    def _(i):
      for k in range(_DIM // _L):
        sl = pl.ds(k * _L, _L)
        x_v[i, sl] = rows_v[i, sl] * _MOM + x_v[i, sl] * (1.0 - _MOM)

    pltpu.sync_copy(x_v, ne_out.at[pl.ds(row0, _CH)])


def _scatter_body(idx_hbm, ne_hbm, mem_ref, idx_v, rows_a, rows_b, sem):
  wid = lax.axis_index("s") * _NC + lax.axis_index("c")
  base = wid * _BPW
  for j in range(_NCH):
    pltpu.sync_copy(idx_hbm.at[pl.ds(base + j * _CH, _CH)], idx_v.at[j])
  rows = (rows_a, rows_b)
  copies = []
  for j in range(_NCH):
    rv = rows[j % 2]
    if j >= 2:
      copies[j - 2].wait()
    pltpu.sync_copy(ne_hbm.at[pl.ds(base + j * _CH, _CH)], rv)
    copies.append(pltpu.async_copy(rv, mem_ref.at[idx_v.at[j]], sem))
  copies[-2].wait()
  copies[-1].wait()


_gather_call = pl.kernel(
    _gather_body,
    out_type=(
        jax.ShapeDtypeStruct((_B, _DIM), jnp.float32),
        jax.ShapeDtypeStruct((_B, _DIM), jnp.float32),
    ),
    mesh=_mesh,
    scratch_types=[
        pltpu.VMEM((_NCH, _CH), jnp.int32),
        pltpu.VMEM((_CH, _DIM), jnp.float32),
        pltpu.VMEM((_CH, _DIM), jnp.float32),
        pltpu.VMEM((_CH, _DIM), jnp.float32),
        pltpu.SemaphoreType.DMA,
        pltpu.SemaphoreType.DMA,
    ],
    compiler_params=pltpu.CompilerParams(use_tc_tiling_on_sc=False),
)

_scatter_call = pl.kernel(
    _scatter_body,
    out_type=(),
    mesh=_mesh,
    scratch_types=[
        pltpu.VMEM((_NCH, _CH), jnp.int32),
        pltpu.VMEM((_CH, _DIM), jnp.float32),
        pltpu.VMEM((_CH, _DIM), jnp.float32),
        pltpu.SemaphoreType.DMA,
    ],
    compiler_params=pltpu.CompilerParams(use_tc_tiling_on_sc=False),
)


@jax.jit
def _run(indices, x, memory):
  mem_t = jnp.transpose(memory)
  mem_rm = _t_fwd(mem_t)
  j = indices % _TBS
  ridx = (indices - j) + 2 * (j % _TBS2) + (j // _TBS2)
  newmem = jax.new_ref(mem_rm.reshape(_NPAD, _DIM))
  da, ne = _gather_call(ridx, x, newmem)
  _scatter_call(ridx, ne, newmem)
  upd = jax.freeze(newmem).reshape(_NPAD // 2, _W)
  out = jnp.transpose(_t_bwd(upd))
  return da, out


def kernel(indices, x, memory):
  return _run(indices, x, memory)


# speedup vs baseline: 1.0084x; 1.0084x over previous
import jax
import jax.numpy as jnp
from jax import lax
from jax.experimental import pallas as pl
from jax.experimental.pallas import tpu as pltpu
from jax.experimental.pallas import tpu_sc as plsc

_SIZE = 1000000
_DIM = 64
_MOM = 0.9
_B = 16384

_NC = 2
_NS = 16
_L = 16
_NW = _NC * _NS
_BPW = _B // _NW
_CH = 128
_NCH = _BPW // _CH

_W = 2 * _DIM
_N = _SIZE + 1
_TBS = 32768
_TBS2 = _TBS // 2
_NB = -(-_N // _TBS)
_NPAD = _NB * _TBS

_mesh = plsc.VectorSubcoreMesh(core_axis_name="c", subcore_axis_name="s")


def _t_fwd_body(src_ref, dst_ref):
  v = src_ref[...]
  dst_ref[:, 0:_DIM] = v[:, 0:_TBS2].T
  dst_ref[:, _DIM:_W] = v[:, _TBS2:_TBS].T


def _t_bwd_body(src_ref, dst_ref):
  v = src_ref[...]
  dst_ref[:, 0:_TBS2] = v[:, 0:_DIM].T
  dst_ref[:, _TBS2:_TBS] = v[:, _DIM:_W].T


_t_fwd = pl.pallas_call(
    _t_fwd_body,
    grid=(_NB,),
    in_specs=[pl.BlockSpec((_DIM, _TBS), lambda i: (0, i))],
    out_specs=pl.BlockSpec((_TBS2, _W), lambda i: (i, 0)),
    out_shape=jax.ShapeDtypeStruct((_NPAD // 2, _W), jnp.float32),
)

_t_bwd = pl.pallas_call(
    _t_bwd_body,
    grid=(_NB,),
    in_specs=[pl.BlockSpec((_TBS2, _W), lambda i: (i, 0))],
    out_specs=pl.BlockSpec((_DIM, _TBS), lambda i: (0, i)),
    out_shape=jax.ShapeDtypeStruct((_DIM, _N), jnp.float32),
)


def _gather_body(idx_hbm, x_hbm, mem_ref, da_out, ne_out, idx_v, rows_a,
                 rows_b, x_v, sem_a, sem_b):
  wid = lax.axis_index("s") * _NC + lax.axis_index("c")
  base = wid * _BPW
  for j in range(_NCH):
    pltpu.sync_copy(idx_hbm.at[pl.ds(base + j * _CH, _CH)], idx_v.at[j])
  rows = (rows_a, rows_b)
  sems = (sem_a, sem_b)
  copies = [pltpu.async_copy(mem_ref.at[idx_v.at[0]], rows[0], sems[0])]
  for j in range(_NCH):
    if j + 1 < _NCH:
      copies.append(pltpu.async_copy(
          mem_ref.at[idx_v.at[j + 1]], rows[(j + 1) % 2], sems[(j + 1) % 2]))
    row0 = base + j * _CH
    pltpu.sync_copy(x_hbm.at[pl.ds(row0, _CH)], x_v)
    copies[j].wait()
    rows_v = rows[j % 2]
    pltpu.sync_copy(rows_v, da_out.at[pl.ds(row0, _CH)])

    @pl.loop(0, _CH)
    def _(i):
      for k in range(_DIM // _L):
        sl = pl.ds(k * _L, _L)
        x_v[i, sl] = rows_v[i, sl] * _MOM + x_v[i, sl] * (1.0 - _MOM)

    pltpu.sync_copy(x_v, ne_out.at[pl.ds(row0, _CH)])


def _scatter_body(idx_hbm, ne_hbm, mem_ref, idx_v, rows_a, rows_b, sem):
  wid = lax.axis_index("s") * _NC + lax.axis_index("c")
  base = wid * _BPW
  for j in range(_NCH):
    pltpu.sync_copy(idx_hbm.at[pl.ds(base + j * _CH, _CH)], idx_v.at[j])
  rows = (rows_a, rows_b)
  copies = []
  for j in range(_NCH):
    rv = rows[j % 2]
    if j >= 2:
      copies[j - 2].wait()
    pltpu.sync_copy(ne_hbm.at[pl.ds(base + j * _CH, _CH)], rv)
    copies.append(pltpu.async_copy(rv, mem_ref.at[idx_v.at[j]], sem))
  copies[-2].wait()
  copies[-1].wait()


_gather_call = pl.kernel(
    _gather_body,
    out_type=(
        jax.ShapeDtypeStruct((_B, _DIM), jnp.float32),
        jax.ShapeDtypeStruct((_B, _DIM), jnp.float32),
    ),
    mesh=_mesh,
    scratch_types=[
        pltpu.VMEM((_NCH, _CH), jnp.int32),
        pltpu.VMEM((_CH, _DIM), jnp.float32),
        pltpu.VMEM((_CH, _DIM), jnp.float32),
        pltpu.VMEM((_CH, _DIM), jnp.float32),
        pltpu.SemaphoreType.DMA,
        pltpu.SemaphoreType.DMA,
    ],
    compiler_params=pltpu.CompilerParams(use_tc_tiling_on_sc=False),
)

_scatter_call = pl.kernel(
    _scatter_body,
    out_type=(),
    mesh=_mesh,
    scratch_types=[
        pltpu.VMEM((_NCH, _CH), jnp.int32),
        pltpu.VMEM((_CH, _DIM), jnp.float32),
        pltpu.VMEM((_CH, _DIM), jnp.float32),
        pltpu.SemaphoreType.DMA,
    ],
    compiler_params=pltpu.CompilerParams(use_tc_tiling_on_sc=False),
)


@jax.jit
def _run(indices, x, memory):
  mem_t = jnp.transpose(memory)
  mem_rm = _t_fwd(mem_t)
  ridx = ((indices & ~(_TBS - 1)) + 2 * (indices & (_TBS2 - 1))
          + ((indices >> (_TBS2.bit_length() - 1)) & 1))
  newmem = jax.new_ref(mem_rm.reshape(_NPAD, _DIM))
  da, ne = _gather_call(ridx, x, newmem)
  _scatter_call(ridx, ne, newmem)
  upd = jax.freeze(newmem).reshape(_NPAD // 2, _W)
  out = jnp.transpose(_t_bwd(upd))
  return da, out


def kernel(indices, x, memory):
  return _run(indices, x, memory)
